# baseline (speedup 1.0000x reference)
import functools

import jax
import jax.numpy as jnp
import numpy as np
from jax import lax
from jax.experimental import pallas as pl
from jax.experimental.pallas import tpu as pltpu
from jax.experimental.pallas import tpu_sc as plsc

B = 4096
L = 200
D = 64
LP = 224
CG = 112
_EPS_BN = 1e-5

_NC = 2
_NS = 16
NW = _NC * _NS
BPW = B // NW

NROWS = 1000000
TBC = 2048
GRID_T = (NROWS + TBC - 1) // TBC
PAIRS = GRID_T * TBC // 2


def _repack_body(tcm_ref, out_ref):
  xt = tcm_ref[:].T
  x3 = xt.reshape(TBC // 2, 2, 64)
  out_ref[:] = jnp.concatenate([x3[:, 0, :], x3[:, 1, :]], axis=1)


def _repack_tc(table_cm):
  return pl.pallas_call(
      _repack_body,
      grid=(GRID_T,),
      in_specs=[pl.BlockSpec((D, TBC), lambda i: (0, i))],
      out_specs=pl.BlockSpec((TBC // 2, 128), lambda i: (i, 0)),
      out_shape=jax.ShapeDtypeStruct((PAIRS, 128), jnp.float32),
  )(table_cm)


def _pool_sc(seq_flat, lens, t2):
  mesh = plsc.VectorSubcoreMesh(core_axis_name="c", subcore_axis_name="s")

  @functools.partial(
      pl.kernel,
      out_type=jax.ShapeDtypeStruct((B, D), jnp.float32),
      mesh=mesh,
      scratch_types=[
          pltpu.VMEM((BPW * LP + 16,), jnp.int32),
          pltpu.VMEM((BPW + 16,), jnp.int32),
          pltpu.VMEM((2 * LP + 16,), jnp.int32),
          pltpu.VMEM((32,), jnp.int32),
          pltpu.VMEM((2, LP, 128), jnp.float32),
          pltpu.VMEM((BPW, D), jnp.float32),
          pltpu.SemaphoreType.DMA,
          pltpu.SemaphoreType.DMA,
      ],
      compiler_params=pltpu.CompilerParams(needs_layout_passes=False,
                                           use_tc_tiling_on_sc=False),
  )
  def k(seq_hbm, lens_hbm, t2_hbm, out_hbm, idx_v, lens_v, comb_v,
        cnt_v, rows_v, out_v, sem0, sem1):
    sems = (sem0, sem1)
    wid = lax.axis_index("s") * _NC + lax.axis_index("c")
    base = wid * BPW
    pltpu.sync_copy(seq_hbm.at[pl.ds(base * LP, BPW * LP)],
                    idx_v.at[pl.ds(0, BPW * LP)])
    pltpu.sync_copy(lens_hbm.at[pl.ds(base, BPW)], lens_v.at[pl.ds(0, BPW)])
    zi = jnp.zeros((16,), jnp.int32)
    for w in range(0, 2 * LP + 16, 16):
      comb_v[pl.ds(w, 16)] = zi

    def row_len(j):
      lv = lens_v[pl.ds(j, 16)]
      return lax.min(lax.max(lv[0], 0), L)

    iota = lax.iota(jnp.int32, 16)

    def occupancy(j, slot):
      n = row_len(j)
      ne = cnt_v[pl.ds(slot * 16, 16)][0]
      oa = ((ne + 15) // 16) * 16
      oe = oa + (n - ne)
      return n, ne, oa, oe

    def fire(j, slot):
      n = row_len(j)
      jb = j * LP
      sbase = slot * LP
      acc = jnp.zeros((16,), jnp.int32)
      for w in range(0, LP, 16):
        iv = idx_v[pl.ds(jb + w, 16)]
        nv = lax.min(lax.max(n - w, 0), 16)
        valid = iota < nv
        acc = acc + (valid & ((iv & 1) == 0)).astype(jnp.int32)
      ne_t = plsc.cumsum(acc)[15]
      cnt_v[pl.ds(slot * 16, 16)] = jnp.full((16,), ne_t, jnp.int32)
      oa = ((ne_t + 15) // 16) * 16
      ne = jnp.int32(0)
      no = jnp.int32(0)
      for w in range(0, LP, 16):
        iv = idx_v[pl.ds(jb + w, 16)]
        nv = lax.min(lax.max(n - w, 0), 16)
        valid = iota < nv
        m_e = valid & ((iv & 1) == 0)
        ce = plsc.cumsum(m_e.astype(jnp.int32))
        cv = jnp.minimum(iota + 1, nv)
        co = cv - ce
        dest = jnp.where(m_e, sbase + ne + ce - 1,
                         jnp.where(valid, sbase + oa + no + co - 1, 2 * LP))
        plsc.store_scatter(comb_v, [dest], iv >> 1)
        new = ce[15]
        ne = ne + new
        no = no + (nv - new)
      end = oa + no
      for c in range(LP // CG):
        @pl.when(end > c * CG)
        def _():
          pltpu.make_async_copy(
              t2_hbm.at[comb_v.at[pl.ds(slot * LP + c * CG, CG)]],
              rows_v.at[slot, pl.ds(c * CG, CG)], sems[slot]).start()

    def drain(j, slot):
      _, _, _, oe = occupancy(j, slot)
      for c in range(LP // CG):
        @pl.when(oe > c * CG)
        def _():
          pltpu.make_async_copy(
              t2_hbm.at[comb_v.at[pl.ds(slot * LP + c * CG, CG)]],
              rows_v.at[slot, pl.ds(c * CG, CG)], sems[slot]).wait()

    def accum(j, slot):
      _, ne, oa, oe = occupancy(j, slot)
      zero = jnp.zeros((16,), jnp.float32)
      accs = (zero,) * 8

      def even_pos(l, accs):
        accs = list(accs)
        for kk in range(4):
          accs[kk] = accs[kk] + rows_v[slot, l, pl.ds(kk * 16, 16)]
        return tuple(accs)

      def odd_pos(l, accs):
        accs = list(accs)
        for kk in range(4):
          accs[kk] = accs[kk] + rows_v[slot, l, pl.ds(64 + kk * 16, 16)]
        return tuple(accs)

      for w in range(LP // 16):
        we = 16 * w
        full = (we + 16 <= ne) | ((we >= oa) & (we + 16 <= oe))
        h = jnp.where(we + 16 <= ne, 0, 64)
        s = 4 * (w % 2)

        def full_body(accs, h=h, we=we, s=s):
          accs = list(accs)
          for t in range(16):
            for kk in range(4):
              accs[s + kk] = accs[s + kk] + rows_v[slot, we + t,
                                                   pl.ds(h + kk * 16, 16)]
          return tuple(accs)

        def edge_body(accs, we=we):
          hi_e = lax.max(we, lax.min(ne, we + 16))
          accs = lax.fori_loop(we, hi_e, even_pos, accs)
          lo_o = lax.max(oa, we)
          hi_o = lax.max(lo_o, lax.min(oe, we + 16))
          return lax.fori_loop(lo_o, hi_o, odd_pos, accs)

        accs = lax.cond(full, full_body, edge_body, accs)

      for kk in range(4):
        out_v[j, pl.ds(kk * 16, 16)] = accs[kk] + accs[kk + 4]

    fire(0, 0)

    def body(g, carry):
      j0 = 2 * g
      fire(j0 + 1, 1)
      drain(j0, 0)
      accum(j0, 0)

      @pl.when(g < BPW // 2 - 1)
      def _():
        fire(j0 + 2, 0)

      drain(j0 + 1, 1)
      accum(j0 + 1, 1)
      return carry

    lax.fori_loop(0, BPW // 2, body, 0)
    pltpu.sync_copy(out_v, out_hbm.at[pl.ds(base, BPW)])

  return k(seq_flat, lens, t2)


_BB = 512


def _mlp_body(sum_ref, lens_ref, w1_ref, b1_ref, w2_ref, b2_ref, w3_ref,
              b3_ref, out_ref):
  lens = jnp.clip(lens_ref[:], 0, L).astype(jnp.float32)
  x = sum_ref[:] / (lens + 1e-9)
  h = jnp.dot(x, w1_ref[:], preferred_element_type=jnp.float32) + b1_ref[:]
  h = jnp.maximum(h, 0.0)
  h = jnp.dot(h, w2_ref[:], preferred_element_type=jnp.float32) + b2_ref[:]
  h = jnp.maximum(h, 0.0)
  o = jnp.dot(h, w3_ref[:], preferred_element_type=jnp.float32) + b3_ref[:]
  n2 = jnp.sum(o * o, axis=1, keepdims=True)
  out_ref[:] = o * lax.rsqrt(jnp.maximum(n2, 1e-24))


def _mlp_tc(psum, lens2d, w1f, b1f, w2f, b2f, w3, b3):
  h1, h2 = w1f.shape[1], w2f.shape[1]
  grid = (B // _BB,)
  return pl.pallas_call(
      _mlp_body,
      grid=grid,
      in_specs=[
          pl.BlockSpec((_BB, D), lambda i: (i, 0)),
          pl.BlockSpec((_BB, 1), lambda i: (i, 0)),
          pl.BlockSpec((D, h1), lambda i: (0, 0)),
          pl.BlockSpec((1, h1), lambda i: (0, 0)),
          pl.BlockSpec((h1, h2), lambda i: (0, 0)),
          pl.BlockSpec((1, h2), lambda i: (0, 0)),
          pl.BlockSpec((h2, D), lambda i: (0, 0)),
          pl.BlockSpec((1, D), lambda i: (0, 0)),
      ],
      out_specs=pl.BlockSpec((_BB, D), lambda i: (i, 0)),
      out_shape=jax.ShapeDtypeStruct((B, D), jnp.float32),
  )(psum, lens2d, w1f, b1f, w2f, b2f, w3, b3)


def kernel(item_sequence, sequence_lengths, table, W1, b1, g1, be1, W2, b2,
           g2, be2, W3, b3):
  seq = item_sequence.astype(jnp.int32)
  seq_flat = jnp.pad(seq, ((0, 0), (0, LP - L))).reshape(B * LP)
  lens = sequence_lengths.astype(jnp.int32)
  t2 = _repack_tc(table.T)
  psum = _pool_sc(seq_flat, lens, t2)
  s = np.float32(1.0 / np.sqrt(1.0 + _EPS_BN))
  w1f = W1 * (g1 * s)[None, :]
  b1f = (b1 * g1 * s + be1).reshape(1, -1)
  w2f = W2 * (g2 * s)[None, :]
  b2f = (b2 * g2 * s + be2).reshape(1, -1)
  return _mlp_tc(psum, lens.reshape(B, 1), w1f, b1f, w2f, b2f, W3,
                 b3.reshape(1, -1))


# speedup vs baseline: 2.7738x; 2.7738x over previous
import functools

import jax
import jax.numpy as jnp
import numpy as np
from jax import lax
from jax.experimental import pallas as pl
from jax.experimental.pallas import tpu as pltpu
from jax.experimental.pallas import tpu_sc as plsc

B = 4096
L = 200
D = 64
CH = 50
NCH = L // CH
_EPS_BN = 1e-5

_NC = 2
_NS = 16
NW = _NC * _NS
BPW = B // NW

NROWS = 1000000
TBC = 1024
GRID_T = (NROWS + TBC - 1) // TBC
TROWS = GRID_T * TBC


def _repack_body(tcm_ref, out_ref):
  xt = tcm_ref[:].T
  out_ref[:, 0:64] = xt
  out_ref[:, 64:128] = xt


def _repack_tc(table_cm):
  return pl.pallas_call(
      _repack_body,
      grid=(GRID_T,),
      in_specs=[pl.BlockSpec((D, TBC), lambda i: (0, i))],
      out_specs=pl.BlockSpec((TBC, 128), lambda i: (i, 0)),
      out_shape=jax.ShapeDtypeStruct((TROWS, 128), jnp.float32),
  )(table_cm)


def _pool_sc(seq, lens, t2):
  mesh = plsc.VectorSubcoreMesh(core_axis_name="c", subcore_axis_name="s")

  @functools.partial(
      pl.kernel,
      out_type=jax.ShapeDtypeStruct((B, D), jnp.float32),
      mesh=mesh,
      scratch_types=[
          pltpu.VMEM((BPW, NCH, CH), jnp.int32),
          pltpu.VMEM((BPW + 16,), jnp.int32),
          pltpu.VMEM((2, NCH, CH, 128), jnp.float32),
          pltpu.VMEM((BPW, D), jnp.float32),
          pltpu.SemaphoreType.DMA,
          pltpu.SemaphoreType.DMA,
      ],
      compiler_params=pltpu.CompilerParams(use_tc_tiling_on_sc=False),
  )
  def k(seq_hbm, lens_hbm, t2_hbm, out_hbm, idx_v, lens_v, rows_v, out_v,
        sem0, sem1):
    sems = (sem0, sem1)
    wid = lax.axis_index("s") * _NC + lax.axis_index("c")
    base = wid * BPW
    pltpu.sync_copy(seq_hbm.at[pl.ds(base, BPW)], idx_v)
    pltpu.sync_copy(lens_hbm.at[pl.ds(base, BPW)], lens_v.at[pl.ds(0, BPW)])

    def row_len(j):
      lv = lens_v[pl.ds(j, 16)]
      return lax.min(lax.max(lv[0], 0), L)

    def fire(j, slot):
      lj = row_len(j)
      for c in range(NCH):
        @pl.when(lj > c * CH)
        def _():
          pltpu.make_async_copy(
              t2_hbm.at[idx_v.at[j, c]], rows_v.at[slot, c],
              sems[slot]).start()

    def drain(j, slot):
      lj = row_len(j)
      for c in range(NCH):
        @pl.when(lj > c * CH)
        def _():
          pltpu.make_async_copy(
              t2_hbm.at[idx_v.at[j, c]], rows_v.at[slot, c],
              sems[slot]).wait()

    def accum(j, slot):
      lj = row_len(j)
      nfull = lj // CH
      nrem = lj - nfull * CH
      zero = jnp.zeros((16,), jnp.float32)
      accs = (zero,) * 8

      def full_chunk(c, accs):
        a = list(accs[:4])
        b = list(accs[4:])
        for l in range(0, CH, 2):
          for kk in range(4):
            a[kk] = a[kk] + rows_v[slot, c, l, pl.ds(kk * 16, 16)]
            b[kk] = b[kk] + rows_v[slot, c, l + 1, pl.ds(kk * 16, 16)]
        return (*a, *b)

      accs = lax.fori_loop(0, nfull, full_chunk, accs)

      def rem_pos(l, accs):
        a = list(accs[:4])
        for kk in range(4):
          a[kk] = a[kk] + rows_v[slot, nfull, l, pl.ds(kk * 16, 16)]
        return (*a, *accs[4:])

      accs = lax.fori_loop(0, nrem, rem_pos, accs)
      for kk in range(4):
        out_v[j, pl.ds(kk * 16, 16)] = accs[kk] + accs[kk + 4]

    fire(0, 0)

    def body(g, carry):
      j0 = 2 * g
      fire(j0 + 1, 1)
      drain(j0, 0)
      accum(j0, 0)

      @pl.when(g < BPW // 2 - 1)
      def _():
        fire(j0 + 2, 0)

      drain(j0 + 1, 1)
      accum(j0 + 1, 1)
      return carry

    lax.fori_loop(0, BPW // 2, body, 0)
    pltpu.sync_copy(out_v, out_hbm.at[pl.ds(base, BPW)])

  return k(seq, lens, t2)


_BB = 512


def _mlp_body(sum_ref, lens_ref, w1_ref, b1_ref, w2_ref, b2_ref, w3_ref,
              b3_ref, out_ref):
  lens = jnp.clip(lens_ref[:], 0, L).astype(jnp.float32)
  x = sum_ref[:] / (lens + 1e-9)
  h = jnp.dot(x, w1_ref[:], preferred_element_type=jnp.float32) + b1_ref[:]
  h = jnp.maximum(h, 0.0)
  h = jnp.dot(h, w2_ref[:], preferred_element_type=jnp.float32) + b2_ref[:]
  h = jnp.maximum(h, 0.0)
  o = jnp.dot(h, w3_ref[:], preferred_element_type=jnp.float32) + b3_ref[:]
  n2 = jnp.sum(o * o, axis=1, keepdims=True)
  out_ref[:] = o * lax.rsqrt(jnp.maximum(n2, 1e-24))


def _mlp_tc(psum, lens2d, w1f, b1f, w2f, b2f, w3, b3):
  h1, h2 = w1f.shape[1], w2f.shape[1]
  grid = (B // _BB,)
  return pl.pallas_call(
      _mlp_body,
      grid=grid,
      in_specs=[
          pl.BlockSpec((_BB, D), lambda i: (i, 0)),
          pl.BlockSpec((_BB, 1), lambda i: (i, 0)),
          pl.BlockSpec((D, h1), lambda i: (0, 0)),
          pl.BlockSpec((1, h1), lambda i: (0, 0)),
          pl.BlockSpec((h1, h2), lambda i: (0, 0)),
          pl.BlockSpec((1, h2), lambda i: (0, 0)),
          pl.BlockSpec((h2, D), lambda i: (0, 0)),
          pl.BlockSpec((1, D), lambda i: (0, 0)),
      ],
      out_specs=pl.BlockSpec((_BB, D), lambda i: (i, 0)),
      out_shape=jax.ShapeDtypeStruct((B, D), jnp.float32),
  )(psum, lens2d, w1f, b1f, w2f, b2f, w3, b3)


def kernel(item_sequence, sequence_lengths, table, W1, b1, g1, be1, W2, b2,
           g2, be2, W3, b3):
  seq = item_sequence.astype(jnp.int32).reshape(B, NCH, CH)
  lens = sequence_lengths.astype(jnp.int32)
  t2 = _repack_tc(table.T)
  psum = _pool_sc(seq, lens, t2)
  s = np.float32(1.0 / np.sqrt(1.0 + _EPS_BN))
  w1f = W1 * (g1 * s)[None, :]
  b1f = (b1 * g1 * s + be1).reshape(1, -1)
  w2f = W2 * (g2 * s)[None, :]
  b2f = (b2 * g2 * s + be2).reshape(1, -1)
  return _mlp_tc(psum, lens.reshape(B, 1), w1f, b1f, w2f, b2f, W3,
                 b3.reshape(1, -1))


# speedup vs baseline: 4.4953x; 1.6206x over previous
import functools

import jax
import jax.numpy as jnp
import numpy as np
from jax import lax
from jax.experimental import pallas as pl
from jax.experimental.pallas import tpu as pltpu
from jax.experimental.pallas import tpu_sc as plsc

B = 4096
L = 200
D = 64
CH = 50
NCH = L // CH
_EPS_BN = 1e-5

_NC = 2
_NS = 16
NW = _NC * _NS
BPW = B // NW

NROWS = 1000000
TBC = 4096
GRID_T = (NROWS + TBC - 1) // TBC
TROWS = GRID_T * TBC


def _repack_body(tcm_ref, out_ref):
  xt = tcm_ref[:].T
  out_ref[:, 0:64] = xt
  out_ref[:, 64:128] = xt


def _repack_tc(table_cm):
  return pl.pallas_call(
      _repack_body,
      grid=(GRID_T,),
      in_specs=[pl.BlockSpec((D, TBC), lambda i: (0, i))],
      out_specs=pl.BlockSpec((TBC, 128), lambda i: (i, 0)),
      out_shape=jax.ShapeDtypeStruct((TROWS, 128), jnp.float32),
  )(table_cm)


def _pool_sc(seq, lens, t2):
  mesh = plsc.VectorSubcoreMesh(core_axis_name="c", subcore_axis_name="s")

  @functools.partial(
      pl.kernel,
      out_type=jax.ShapeDtypeStruct((B, D), jnp.float32),
      mesh=mesh,
      scratch_types=[
          pltpu.VMEM((BPW, NCH, CH), jnp.int32),
          pltpu.VMEM((BPW + 16,), jnp.int32),
          pltpu.VMEM((2, NCH, CH, 128), jnp.float32),
          pltpu.VMEM((BPW, D), jnp.float32),
          pltpu.SemaphoreType.DMA,
          pltpu.SemaphoreType.DMA,
      ],
      compiler_params=pltpu.CompilerParams(use_tc_tiling_on_sc=False),
  )
  def k(seq_hbm, lens_hbm, t2_hbm, out_hbm, idx_v, lens_v, rows_v, out_v,
        sem0, sem1):
    sems = (sem0, sem1)
    wid = lax.axis_index("s") * _NC + lax.axis_index("c")
    base = wid * BPW
    pltpu.sync_copy(seq_hbm.at[pl.ds(base, BPW)], idx_v)
    pltpu.sync_copy(lens_hbm.at[pl.ds(base, BPW)], lens_v.at[pl.ds(0, BPW)])

    def row_len(j):
      lv = lens_v[pl.ds(j, 16)]
      return lax.min(lax.max(lv[0], 0), L)

    def fire(j, slot):
      lj = row_len(j)
      for c in range(NCH):
        @pl.when(lj > c * CH)
        def _():
          pltpu.make_async_copy(
              t2_hbm.at[idx_v.at[j, c]], rows_v.at[slot, c],
              sems[slot]).start()

    def drain(j, slot):
      lj = row_len(j)
      for c in range(NCH):
        @pl.when(lj > c * CH)
        def _():
          pltpu.make_async_copy(
              t2_hbm.at[idx_v.at[j, c]], rows_v.at[slot, c],
              sems[slot]).wait()

    def accum(j, slot):
      lj = row_len(j)
      nfull = lj // CH
      nrem = lj - nfull * CH
      zero = jnp.zeros((16,), jnp.float32)
      accs = (zero,) * 8

      def full_chunk(c, accs):
        a = list(accs[:4])
        b = list(accs[4:])
        for l in range(0, CH, 2):
          for kk in range(4):
            a[kk] = a[kk] + rows_v[slot, c, l, pl.ds(kk * 16, 16)]
            b[kk] = b[kk] + rows_v[slot, c, l + 1, pl.ds(kk * 16, 16)]
        return (*a, *b)

      accs = lax.fori_loop(0, nfull, full_chunk, accs)

      def rem_pos(l, accs):
        a = list(accs[:4])
        for kk in range(4):
          a[kk] = a[kk] + rows_v[slot, nfull, l, pl.ds(kk * 16, 16)]
        return (*a, *accs[4:])

      accs = lax.fori_loop(0, nrem, rem_pos, accs)
      for kk in range(4):
        out_v[j, pl.ds(kk * 16, 16)] = accs[kk] + accs[kk + 4]

    fire(0, 0)

    def body(g, carry):
      j0 = 2 * g
      fire(j0 + 1, 1)
      drain(j0, 0)
      accum(j0, 0)

      @pl.when(g < BPW // 2 - 1)
      def _():
        fire(j0 + 2, 0)

      drain(j0 + 1, 1)
      accum(j0 + 1, 1)
      return carry

    lax.fori_loop(0, BPW // 2, body, 0)
    pltpu.sync_copy(out_v, out_hbm.at[pl.ds(base, BPW)])

  return k(seq, lens, t2)


_BB = 512


def _mlp_body(sum_ref, lens_ref, w1_ref, b1_ref, w2_ref, b2_ref, w3_ref,
              b3_ref, out_ref):
  lens = jnp.clip(lens_ref[:], 0, L).astype(jnp.float32)
  x = sum_ref[:] / (lens + 1e-9)
  h = jnp.dot(x, w1_ref[:], preferred_element_type=jnp.float32) + b1_ref[:]
  h = jnp.maximum(h, 0.0)
  h = jnp.dot(h, w2_ref[:], preferred_element_type=jnp.float32) + b2_ref[:]
  h = jnp.maximum(h, 0.0)
  o = jnp.dot(h, w3_ref[:], preferred_element_type=jnp.float32) + b3_ref[:]
  n2 = jnp.sum(o * o, axis=1, keepdims=True)
  out_ref[:] = o * lax.rsqrt(jnp.maximum(n2, 1e-24))


def _mlp_tc(psum, lens2d, w1f, b1f, w2f, b2f, w3, b3):
  h1, h2 = w1f.shape[1], w2f.shape[1]
  grid = (B // _BB,)
  return pl.pallas_call(
      _mlp_body,
      grid=grid,
      in_specs=[
          pl.BlockSpec((_BB, D), lambda i: (i, 0)),
          pl.BlockSpec((_BB, 1), lambda i: (i, 0)),
          pl.BlockSpec((D, h1), lambda i: (0, 0)),
          pl.BlockSpec((1, h1), lambda i: (0, 0)),
          pl.BlockSpec((h1, h2), lambda i: (0, 0)),
          pl.BlockSpec((1, h2), lambda i: (0, 0)),
          pl.BlockSpec((h2, D), lambda i: (0, 0)),
          pl.BlockSpec((1, D), lambda i: (0, 0)),
      ],
      out_specs=pl.BlockSpec((_BB, D), lambda i: (i, 0)),
      out_shape=jax.ShapeDtypeStruct((B, D), jnp.float32),
  )(psum, lens2d, w1f, b1f, w2f, b2f, w3, b3)


def kernel(item_sequence, sequence_lengths, table, W1, b1, g1, be1, W2, b2,
           g2, be2, W3, b3):
  seq = item_sequence.astype(jnp.int32).reshape(B, NCH, CH)
  lens = sequence_lengths.astype(jnp.int32)
  t2 = _repack_tc(table.T)
  psum = _pool_sc(seq, lens, t2)
  s = np.float32(1.0 / np.sqrt(1.0 + _EPS_BN))
  w1f = W1 * (g1 * s)[None, :]
  b1f = (b1 * g1 * s + be1).reshape(1, -1)
  w2f = W2 * (g2 * s)[None, :]
  b2f = (b2 * g2 * s + be2).reshape(1, -1)
  return _mlp_tc(psum, lens.reshape(B, 1), w1f, b1f, w2f, b2f, W3,
                 b3.reshape(1, -1))


# speedup vs baseline: 5.4163x; 1.2049x over previous
import functools

import jax
import jax.numpy as jnp
import numpy as np
from jax import lax
from jax.experimental import pallas as pl
from jax.experimental.pallas import tpu as pltpu
from jax.experimental.pallas import tpu_sc as plsc

B = 4096
L = 200
D = 64
CH = 50
NCH = L // CH
_EPS_BN = 1e-5

_NC = 2
_NS = 16
NW = _NC * _NS
BPW = B // NW

NROWS = 1000000
TBC = 16384
GRID_T = (NROWS + TBC - 1) // TBC
TROWS = GRID_T * TBC


def _repack_body(tcm_ref, out_ref):
  xt = tcm_ref[:].T
  out_ref[:, 0:64] = xt
  out_ref[:, 64:128] = xt


def _repack_tc(table_cm):
  return pl.pallas_call(
      _repack_body,
      grid=(GRID_T,),
      in_specs=[pl.BlockSpec((D, TBC), lambda i: (0, i))],
      out_specs=pl.BlockSpec((TBC, 128), lambda i: (i, 0)),
      out_shape=jax.ShapeDtypeStruct((TROWS, 128), jnp.float32),
  )(table_cm)


def _pool_sc(seq, lens, t2):
  mesh = plsc.VectorSubcoreMesh(core_axis_name="c", subcore_axis_name="s")

  @functools.partial(
      pl.kernel,
      out_type=jax.ShapeDtypeStruct((B, D), jnp.float32),
      mesh=mesh,
      scratch_types=[
          pltpu.VMEM((BPW, NCH, CH), jnp.int32),
          pltpu.VMEM((BPW + 16,), jnp.int32),
          pltpu.VMEM((2, NCH, CH, 128), jnp.float32),
          pltpu.VMEM((BPW, D), jnp.float32),
          pltpu.SemaphoreType.DMA,
          pltpu.SemaphoreType.DMA,
      ],
      compiler_params=pltpu.CompilerParams(use_tc_tiling_on_sc=False),
  )
  def k(seq_hbm, lens_hbm, t2_hbm, out_hbm, idx_v, lens_v, rows_v, out_v,
        sem0, sem1):
    sems = (sem0, sem1)
    wid = lax.axis_index("s") * _NC + lax.axis_index("c")
    base = wid * BPW
    pltpu.sync_copy(seq_hbm.at[pl.ds(base, BPW)], idx_v)
    pltpu.sync_copy(lens_hbm.at[pl.ds(base, BPW)], lens_v.at[pl.ds(0, BPW)])

    def row_len(j):
      lv = lens_v[pl.ds(j, 16)]
      return lax.min(lax.max(lv[0], 0), L)

    def fire(j, slot):
      lj = row_len(j)
      for c in range(NCH):
        @pl.when(lj > c * CH)
        def _():
          pltpu.make_async_copy(
              t2_hbm.at[idx_v.at[j, c]], rows_v.at[slot, c],
              sems[slot]).start()

    def drain(j, slot):
      lj = row_len(j)
      for c in range(NCH):
        @pl.when(lj > c * CH)
        def _():
          pltpu.make_async_copy(
              t2_hbm.at[idx_v.at[j, c]], rows_v.at[slot, c],
              sems[slot]).wait()

    def accum(j, slot):
      lj = row_len(j)
      nfull = lj // CH
      nrem = lj - nfull * CH
      zero = jnp.zeros((16,), jnp.float32)
      accs = (zero,) * 8

      def full_chunk(c, accs):
        a = list(accs[:4])
        b = list(accs[4:])
        for l in range(0, CH, 2):
          for kk in range(4):
            a[kk] = a[kk] + rows_v[slot, c, l, pl.ds(kk * 16, 16)]
            b[kk] = b[kk] + rows_v[slot, c, l + 1, pl.ds(kk * 16, 16)]
        return (*a, *b)

      accs = lax.fori_loop(0, nfull, full_chunk, accs)

      def rem_pos(l, accs):
        a = list(accs[:4])
        for kk in range(4):
          a[kk] = a[kk] + rows_v[slot, nfull, l, pl.ds(kk * 16, 16)]
        return (*a, *accs[4:])

      accs = lax.fori_loop(0, nrem, rem_pos, accs)
      for kk in range(4):
        out_v[j, pl.ds(kk * 16, 16)] = accs[kk] + accs[kk + 4]

    fire(0, 0)

    def body(g, carry):
      j0 = 2 * g
      fire(j0 + 1, 1)
      drain(j0, 0)
      accum(j0, 0)

      @pl.when(g < BPW // 2 - 1)
      def _():
        fire(j0 + 2, 0)

      drain(j0 + 1, 1)
      accum(j0 + 1, 1)
      return carry

    lax.fori_loop(0, BPW // 2, body, 0)
    pltpu.sync_copy(out_v, out_hbm.at[pl.ds(base, BPW)])

  return k(seq, lens, t2)


_BB = 512


def _mlp_body(sum_ref, lens_ref, w1_ref, b1_ref, w2_ref, b2_ref, w3_ref,
              b3_ref, out_ref):
  lens = jnp.clip(lens_ref[:], 0, L).astype(jnp.float32)
  x = sum_ref[:] / (lens + 1e-9)
  h = jnp.dot(x, w1_ref[:], preferred_element_type=jnp.float32) + b1_ref[:]
  h = jnp.maximum(h, 0.0)
  h = jnp.dot(h, w2_ref[:], preferred_element_type=jnp.float32) + b2_ref[:]
  h = jnp.maximum(h, 0.0)
  o = jnp.dot(h, w3_ref[:], preferred_element_type=jnp.float32) + b3_ref[:]
  n2 = jnp.sum(o * o, axis=1, keepdims=True)
  out_ref[:] = o * lax.rsqrt(jnp.maximum(n2, 1e-24))


def _mlp_tc(psum, lens2d, w1f, b1f, w2f, b2f, w3, b3):
  h1, h2 = w1f.shape[1], w2f.shape[1]
  grid = (B // _BB,)
  return pl.pallas_call(
      _mlp_body,
      grid=grid,
      in_specs=[
          pl.BlockSpec((_BB, D), lambda i: (i, 0)),
          pl.BlockSpec((_BB, 1), lambda i: (i, 0)),
          pl.BlockSpec((D, h1), lambda i: (0, 0)),
          pl.BlockSpec((1, h1), lambda i: (0, 0)),
          pl.BlockSpec((h1, h2), lambda i: (0, 0)),
          pl.BlockSpec((1, h2), lambda i: (0, 0)),
          pl.BlockSpec((h2, D), lambda i: (0, 0)),
          pl.BlockSpec((1, D), lambda i: (0, 0)),
      ],
      out_specs=pl.BlockSpec((_BB, D), lambda i: (i, 0)),
      out_shape=jax.ShapeDtypeStruct((B, D), jnp.float32),
  )(psum, lens2d, w1f, b1f, w2f, b2f, w3, b3)


def kernel(item_sequence, sequence_lengths, table, W1, b1, g1, be1, W2, b2,
           g2, be2, W3, b3):
  seq = item_sequence.astype(jnp.int32).reshape(B, NCH, CH)
  lens = sequence_lengths.astype(jnp.int32)
  t2 = _repack_tc(table.T)
  psum = _pool_sc(seq, lens, t2)
  s = np.float32(1.0 / np.sqrt(1.0 + _EPS_BN))
  w1f = W1 * (g1 * s)[None, :]
  b1f = (b1 * g1 * s + be1).reshape(1, -1)
  w2f = W2 * (g2 * s)[None, :]
  b2f = (b2 * g2 * s + be2).reshape(1, -1)
  return _mlp_tc(psum, lens.reshape(B, 1), w1f, b1f, w2f, b2f, W3,
                 b3.reshape(1, -1))


# speedup vs baseline: 5.5327x; 1.0215x over previous
import functools

import jax
import jax.numpy as jnp
import numpy as np
from jax import lax
from jax.experimental import pallas as pl
from jax.experimental.pallas import tpu as pltpu
from jax.experimental.pallas import tpu_sc as plsc

B = 4096
L = 200
D = 64
CH = 50
NCH = L // CH
_EPS_BN = 1e-5

_NC = 2
_NS = 16
NW = _NC * _NS
BPW = B // NW

NROWS = 1000000
TBC = 20480
GRID_T = (NROWS + TBC - 1) // TBC
TROWS = GRID_T * TBC


def _repack_body(tcm_ref, out_ref):
  xt = tcm_ref[:].T
  out_ref[:, 0:64] = xt
  out_ref[:, 64:128] = xt


def _repack_tc(table_cm):
  return pl.pallas_call(
      _repack_body,
      grid=(GRID_T,),
      in_specs=[pl.BlockSpec((D, TBC), lambda i: (0, i))],
      out_specs=pl.BlockSpec((TBC, 128), lambda i: (i, 0)),
      out_shape=jax.ShapeDtypeStruct((TROWS, 128), jnp.float32),
  )(table_cm)


def _pool_sc(seq, lens, t2):
  mesh = plsc.VectorSubcoreMesh(core_axis_name="c", subcore_axis_name="s")

  @functools.partial(
      pl.kernel,
      out_type=jax.ShapeDtypeStruct((B, D), jnp.float32),
      mesh=mesh,
      scratch_types=[
          pltpu.VMEM((BPW, NCH, CH), jnp.int32),
          pltpu.VMEM((BPW + 16,), jnp.int32),
          pltpu.VMEM((2, NCH, CH, 128), jnp.float32),
          pltpu.VMEM((BPW, D), jnp.float32),
          pltpu.SemaphoreType.DMA,
          pltpu.SemaphoreType.DMA,
      ],
      compiler_params=pltpu.CompilerParams(use_tc_tiling_on_sc=False),
  )
  def k(seq_hbm, lens_hbm, t2_hbm, out_hbm, idx_v, lens_v, rows_v, out_v,
        sem0, sem1):
    sems = (sem0, sem1)
    wid = lax.axis_index("s") * _NC + lax.axis_index("c")
    base = wid * BPW
    pltpu.sync_copy(seq_hbm.at[pl.ds(base, BPW)], idx_v)
    pltpu.sync_copy(lens_hbm.at[pl.ds(base, BPW)], lens_v.at[pl.ds(0, BPW)])

    def row_len(j):
      lv = lens_v[pl.ds(j, 16)]
      return lax.min(lax.max(lv[0], 0), L)

    def fire(j, slot):
      lj = row_len(j)
      for c in range(NCH):
        @pl.when(lj > c * CH)
        def _():
          pltpu.make_async_copy(
              t2_hbm.at[idx_v.at[j, c]], rows_v.at[slot, c],
              sems[slot]).start()

    def drain(j, slot):
      lj = row_len(j)
      for c in range(NCH):
        @pl.when(lj > c * CH)
        def _():
          pltpu.make_async_copy(
              t2_hbm.at[idx_v.at[j, c]], rows_v.at[slot, c],
              sems[slot]).wait()

    def accum(j, slot):
      lj = row_len(j)
      nfull = lj // CH
      nrem = lj - nfull * CH
      zero = jnp.zeros((16,), jnp.float32)
      accs = (zero,) * 8

      def full_chunk(c, accs):
        a = list(accs[:4])
        b = list(accs[4:])
        for l in range(0, CH, 2):
          for kk in range(4):
            a[kk] = a[kk] + rows_v[slot, c, l, pl.ds(kk * 16, 16)]
            b[kk] = b[kk] + rows_v[slot, c, l + 1, pl.ds(kk * 16, 16)]
        return (*a, *b)

      accs = lax.fori_loop(0, nfull, full_chunk, accs)

      def rem_pos(l, accs):
        a = list(accs[:4])
        for kk in range(4):
          a[kk] = a[kk] + rows_v[slot, nfull, l, pl.ds(kk * 16, 16)]
        return (*a, *accs[4:])

      accs = lax.fori_loop(0, nrem, rem_pos, accs)
      for kk in range(4):
        out_v[j, pl.ds(kk * 16, 16)] = accs[kk] + accs[kk + 4]

    fire(0, 0)

    def body(g, carry):
      j0 = 2 * g
      fire(j0 + 1, 1)
      drain(j0, 0)
      accum(j0, 0)

      @pl.when(g < BPW // 2 - 1)
      def _():
        fire(j0 + 2, 0)

      drain(j0 + 1, 1)
      accum(j0 + 1, 1)
      return carry

    lax.fori_loop(0, BPW // 2, body, 0)
    pltpu.sync_copy(out_v, out_hbm.at[pl.ds(base, BPW)])

  return k(seq, lens, t2)


_BB = 512


def _mlp_body(sum_ref, lens_ref, w1_ref, b1_ref, w2_ref, b2_ref, w3_ref,
              b3_ref, out_ref):
  lens = jnp.clip(lens_ref[:], 0, L).astype(jnp.float32)
  x = sum_ref[:] / (lens + 1e-9)
  h = jnp.dot(x, w1_ref[:], preferred_element_type=jnp.float32) + b1_ref[:]
  h = jnp.maximum(h, 0.0)
  h = jnp.dot(h, w2_ref[:], preferred_element_type=jnp.float32) + b2_ref[:]
  h = jnp.maximum(h, 0.0)
  o = jnp.dot(h, w3_ref[:], preferred_element_type=jnp.float32) + b3_ref[:]
  n2 = jnp.sum(o * o, axis=1, keepdims=True)
  out_ref[:] = o * lax.rsqrt(jnp.maximum(n2, 1e-24))


def _mlp_tc(psum, lens2d, w1f, b1f, w2f, b2f, w3, b3):
  h1, h2 = w1f.shape[1], w2f.shape[1]
  grid = (B // _BB,)
  return pl.pallas_call(
      _mlp_body,
      grid=grid,
      in_specs=[
          pl.BlockSpec((_BB, D), lambda i: (i, 0)),
          pl.BlockSpec((_BB, 1), lambda i: (i, 0)),
          pl.BlockSpec((D, h1), lambda i: (0, 0)),
          pl.BlockSpec((1, h1), lambda i: (0, 0)),
          pl.BlockSpec((h1, h2), lambda i: (0, 0)),
          pl.BlockSpec((1, h2), lambda i: (0, 0)),
          pl.BlockSpec((h2, D), lambda i: (0, 0)),
          pl.BlockSpec((1, D), lambda i: (0, 0)),
      ],
      out_specs=pl.BlockSpec((_BB, D), lambda i: (i, 0)),
      out_shape=jax.ShapeDtypeStruct((B, D), jnp.float32),
  )(psum, lens2d, w1f, b1f, w2f, b2f, w3, b3)


def kernel(item_sequence, sequence_lengths, table, W1, b1, g1, be1, W2, b2,
           g2, be2, W3, b3):
  seq = item_sequence.astype(jnp.int32).reshape(B, NCH, CH)
  lens = sequence_lengths.astype(jnp.int32)
  t2 = _repack_tc(table.T)
  psum = _pool_sc(seq, lens, t2)
  s = np.float32(1.0 / np.sqrt(1.0 + _EPS_BN))
  w1f = W1 * (g1 * s)[None, :]
  b1f = (b1 * g1 * s + be1).reshape(1, -1)
  w2f = W2 * (g2 * s)[None, :]
  b2f = (b2 * g2 * s + be2).reshape(1, -1)
  return _mlp_tc(psum, lens.reshape(B, 1), w1f, b1f, w2f, b2f, W3,
                 b3.reshape(1, -1))
